# baseline (speedup 1.0000x reference)
import dataclasses
import functools

import jax
import jax.numpy as jnp
from jax import lax
from jax.experimental import pallas as pl
from jax.experimental.pallas import tpu as pltpu
from jax.experimental.pallas import tpu_sc as plsc

_TOP_K = 8
_N_EXP = 64
_ROWS_PER_BLOCK = 1024
_N_TILES = 32
_SC_CORES = 2
_N_CHUNKS = 4


def _matmul_block(hs2_ref, gw_ref, flat_ref):
    hs2 = hs2_ref[...]
    gw = gw_ref[...]
    dim = gw.shape[1]
    dn = (((1,), (1,)), ((), ()))
    even = jax.lax.dot_general(
        hs2[:, 0:dim], gw, dn, preferred_element_type=jnp.float32)
    odd = jax.lax.dot_general(
        hs2[:, dim:2 * dim], gw, dn, preferred_element_type=jnp.float32)
    flat_ref[...] = jnp.concatenate([even, odd], axis=1)


def _matmul_chunk(hidden2, gate_weight, chunk, n_chunks):
    half_rows, two_dim = hidden2.shape
    tokens, dim = half_rows * 2, two_dim // 2
    n_exp = gate_weight.shape[0]
    rows_c = tokens // n_chunks
    r = min(_ROWS_PER_BLOCK, rows_c)
    blk0 = chunk * (rows_c // (2 * (r // 2)))
    return pl.pallas_call(
        _matmul_block,
        grid=(rows_c // r,),
        in_specs=[
            pl.BlockSpec((r // 2, two_dim), lambda b: (b + blk0, 0)),
            pl.BlockSpec((n_exp, dim), lambda b: (0, 0)),
        ],
        out_specs=pl.BlockSpec((r // 2, 128), lambda b: (b, 0)),
        out_shape=jax.ShapeDtypeStruct((rows_c // 2, 128), jnp.float32),
        compiler_params=pltpu.CompilerParams(
            dimension_semantics=("arbitrary",),
        ),
    )(hidden2, gate_weight)


def _make_topk_sc(tokens):
    rpt = tokens // _N_TILES
    mesh = plsc.VectorSubcoreMesh(core_axis_name="c", subcore_axis_name="s")
    cp = pltpu.CompilerParams()
    if "needs_layout_passes" in pltpu.CompilerParams.__dataclass_fields__:
        cp = dataclasses.replace(cp, needs_layout_passes=False)

    @functools.partial(
        pl.kernel,
        out_type=(
            jax.ShapeDtypeStruct((tokens * _TOP_K,), jnp.float32),
            jax.ShapeDtypeStruct((tokens * _TOP_K,), jnp.int32),
        ),
        mesh=mesh,
        scratch_types=[
            pltpu.VMEM((rpt * _N_EXP,), jnp.float32),
            pltpu.VMEM((rpt * _TOP_K + 16,), jnp.float32),
            pltpu.VMEM((rpt * _TOP_K + 16,), jnp.int32),
        ],
        compiler_params=cp,
    )
    def topk_kernel(logits_hbm, w_hbm, i_hbm, lv, wv, iv):
        wid = lax.axis_index("s") * _SC_CORES + lax.axis_index("c")
        base = wid * rpt
        pltpu.sync_copy(logits_hbm.at[pl.ds(base * _N_EXP, rpt * _N_EXP)], lv)

        lanes = lax.iota(jnp.int32, 16)
        low = lanes < 8

        def merge(ak, av, bk, bv):
            mk = jnp.where(low, ak, lax.rev(bk, (0,)))
            mv = jnp.where(low, av, lax.rev(bv, (0,)))
            return plsc.sort_key_val(mk, mv, descending=True)

        @pl.loop(0, rpt)
        def _row(r):
            rbase = r * _N_EXP
            ks, vs = [], []
            for j in range(4):
                c = lv[pl.ds(rbase + 16 * j, 16)]
                sk, sv = plsc.sort_key_val(c, lanes + (16 * j),
                                           descending=True)
                ks.append(sk)
                vs.append(sv)
            abk, abv = merge(ks[0], vs[0], ks[1], vs[1])
            cdk, cdv = merge(ks[2], vs[2], ks[3], vs[3])
            k8, i8 = merge(abk, abv, cdk, cdv)

            m = jnp.max(k8)
            e = jnp.exp(k8 - m)
            den = jnp.sum(jnp.where(low, e, 0.0))
            w = e / den
            plsc.store_compressed(wv.at[pl.ds(r * _TOP_K, 16)], w, mask=low)
            plsc.store_compressed(iv.at[pl.ds(r * _TOP_K, 16)], i8, mask=low)

        pltpu.sync_copy(wv.at[pl.ds(0, rpt * _TOP_K)],
                        w_hbm.at[pl.ds(base * _TOP_K, rpt * _TOP_K)])
        pltpu.sync_copy(iv.at[pl.ds(0, rpt * _TOP_K)],
                        i_hbm.at[pl.ds(base * _TOP_K, rpt * _TOP_K)])

    return topk_kernel


@jax.jit
def kernel(hidden_states, gate_weight):
    tokens, dim = hidden_states.shape
    rows_c = tokens // _N_CHUNKS
    hidden2 = hidden_states.reshape(tokens // 2, 2 * dim)
    topk = _make_topk_sc(rows_c)
    flat_parts, w_parts, i_parts = [], [], []
    for c in range(_N_CHUNKS):
        flat = _matmul_chunk(hidden2, gate_weight, c, _N_CHUNKS)
        w_f, i_f = topk(flat.reshape(-1))
        flat_parts.append(flat)
        w_parts.append(w_f)
        i_parts.append(i_f)
    logits = jnp.concatenate(flat_parts, axis=0).reshape(tokens, _N_EXP)
    w = jnp.concatenate(w_parts, axis=0).reshape(tokens, _TOP_K)
    i = jnp.concatenate(i_parts, axis=0).reshape(tokens, _TOP_K)
    return (w, i, logits)


# speedup vs baseline: 1.9880x; 1.9880x over previous
import dataclasses
import functools

import jax
import jax.numpy as jnp
from jax import lax
from jax.experimental import pallas as pl
from jax.experimental.pallas import tpu as pltpu
from jax.experimental.pallas import tpu_sc as plsc

_TOP_K = 8
_N_EXP = 64
_ROWS_PER_BLOCK = 1024
_N_TILES = 32
_SC_CORES = 2
_N_CHUNKS = 4


def _matmul_block(hs_ref, gw_ref, flat_ref):
    hs = hs_ref[...]
    gw = gw_ref[...]
    r, dim = hs.shape
    dn = (((1,), (1,)), ((), ()))
    top = jax.lax.dot_general(
        hs[0:r // 2, :], gw, dn, preferred_element_type=jnp.float32)
    bot = jax.lax.dot_general(
        hs[r // 2:r, :], gw, dn, preferred_element_type=jnp.float32)
    flat_ref[...] = jnp.concatenate([top, bot], axis=1)


def _matmul_chunk(hidden_states, gate_weight, chunk, n_chunks):
    tokens, dim = hidden_states.shape
    n_exp = gate_weight.shape[0]
    rows_c = tokens // n_chunks
    r = min(_ROWS_PER_BLOCK, rows_c)
    blk0 = chunk * (rows_c // r)
    return pl.pallas_call(
        _matmul_block,
        grid=(rows_c // r,),
        in_specs=[
            pl.BlockSpec((r, dim), lambda b: (b + blk0, 0)),
            pl.BlockSpec((n_exp, dim), lambda b: (0, 0)),
        ],
        out_specs=pl.BlockSpec((r // 2, 2 * n_exp), lambda b: (b, 0)),
        out_shape=jax.ShapeDtypeStruct((rows_c // 2, 2 * n_exp), jnp.float32),
        compiler_params=pltpu.CompilerParams(
            dimension_semantics=("arbitrary",),
        ),
    )(hidden_states, gate_weight)


def _make_topk_sc(rows_c, r_block):
    half = r_block // 2
    rows_flat = rows_c // 2
    tpw = rows_flat // _N_TILES
    mesh = plsc.VectorSubcoreMesh(core_axis_name="c", subcore_axis_name="s")
    cp = pltpu.CompilerParams()
    if "needs_layout_passes" in pltpu.CompilerParams.__dataclass_fields__:
        cp = dataclasses.replace(cp, needs_layout_passes=False)

    @functools.partial(
        pl.kernel,
        out_type=(
            jax.ShapeDtypeStruct((rows_c * _TOP_K,), jnp.float32),
            jax.ShapeDtypeStruct((rows_c * _TOP_K,), jnp.int32),
        ),
        mesh=mesh,
        scratch_types=[
            pltpu.VMEM((tpw * 128,), jnp.float32),
            pltpu.VMEM((2 * tpw * _TOP_K + 16,), jnp.float32),
            pltpu.VMEM((2 * tpw * _TOP_K + 16,), jnp.int32),
        ],
        compiler_params=cp,
    )
    def topk_kernel(flat_hbm, w_hbm, i_hbm, lv, wv, iv):
        wid = lax.axis_index("s") * _SC_CORES + lax.axis_index("c")
        fbase = wid * tpw
        pltpu.sync_copy(flat_hbm.at[pl.ds(fbase * 128, tpw * 128)], lv)

        blk = fbase // half
        tok_top = blk * r_block + (fbase - blk * half)
        tok_bot = tok_top + half

        lanes = lax.iota(jnp.int32, 16)
        low = lanes < 8

        def merge(ak, av, bk, bv):
            mk = jnp.where(low, ak, lax.rev(bk, (0,)))
            mv = jnp.where(low, av, lax.rev(bv, (0,)))
            return plsc.sort_key_val(mk, mv, descending=True)

        @pl.loop(0, tpw)
        def _row(r):
            rbase = r * 128
            for h in range(2):
                ks, vs = [], []
                for j in range(4):
                    c = lv[pl.ds(rbase + h * 64 + 16 * j, 16)]
                    sk, sv = plsc.sort_key_val(c, lanes + (16 * j),
                                               descending=True)
                    ks.append(sk)
                    vs.append(sv)
                abk, abv = merge(ks[0], vs[0], ks[1], vs[1])
                cdk, cdv = merge(ks[2], vs[2], ks[3], vs[3])
                k8, i8 = merge(abk, abv, cdk, cdv)

                m = jnp.max(k8)
                e = jnp.exp(k8 - m)
                den = jnp.sum(jnp.where(low, e, 0.0))
                w = e / den
                out = (h * tpw + r) * _TOP_K
                plsc.store_compressed(wv.at[pl.ds(out, 16)], w, mask=low)
                plsc.store_compressed(iv.at[pl.ds(out, 16)], i8, mask=low)

        n = tpw * _TOP_K
        pltpu.sync_copy(wv.at[pl.ds(0, n)],
                        w_hbm.at[pl.ds(tok_top * _TOP_K, n)])
        pltpu.sync_copy(iv.at[pl.ds(0, n)],
                        i_hbm.at[pl.ds(tok_top * _TOP_K, n)])
        pltpu.sync_copy(wv.at[pl.ds(n, n)],
                        w_hbm.at[pl.ds(tok_bot * _TOP_K, n)])
        pltpu.sync_copy(iv.at[pl.ds(n, n)],
                        i_hbm.at[pl.ds(tok_bot * _TOP_K, n)])

    return topk_kernel


@jax.jit
def kernel(hidden_states, gate_weight):
    tokens, dim = hidden_states.shape
    rows_c = tokens // _N_CHUNKS
    r = min(_ROWS_PER_BLOCK, rows_c)
    topk = _make_topk_sc(rows_c, r)
    flat_parts, w_parts, i_parts = [], [], []
    for c in range(_N_CHUNKS):
        flat = _matmul_chunk(hidden_states, gate_weight, c, _N_CHUNKS)
        w_f, i_f = topk(flat.reshape(-1))
        flat_parts.append(flat)
        w_parts.append(w_f)
        i_parts.append(i_f)
    ft = jnp.concatenate(flat_parts, axis=0)
    ft = ft.reshape(tokens // r, r // 2, 2, _N_EXP)
    logits = ft.transpose(0, 2, 1, 3).reshape(tokens, _N_EXP)
    w = jnp.concatenate(w_parts, axis=0).reshape(tokens, _TOP_K)
    i = jnp.concatenate(i_parts, axis=0).reshape(tokens, _TOP_K)
    return (w, i, logits)


# speedup vs baseline: 1.9971x; 1.0046x over previous
import dataclasses
import functools

import jax
import jax.numpy as jnp
from jax import lax
from jax.experimental import pallas as pl
from jax.experimental.pallas import tpu as pltpu
from jax.experimental.pallas import tpu_sc as plsc

_TOP_K = 8
_N_EXP = 64
_ROWS_PER_BLOCK = 1024
_N_TILES = 32
_SC_CORES = 2
_N_CHUNKS = 4


def _matmul_block(hs_ref, gw_ref, logits_ref, flat_ref):
    hs = hs_ref[...]
    gw = gw_ref[...]
    r, dim = hs.shape
    dn = (((1,), (1,)), ((), ()))
    top = jax.lax.dot_general(
        hs[0:r // 2, :], gw, dn, preferred_element_type=jnp.float32)
    bot = jax.lax.dot_general(
        hs[r // 2:r, :], gw, dn, preferred_element_type=jnp.float32)
    logits_ref[...] = jnp.concatenate([top, bot], axis=0)
    flat_ref[...] = jnp.concatenate([top, bot], axis=1)


def _matmul_chunk(hidden_states, gate_weight, chunk, n_chunks):
    tokens, dim = hidden_states.shape
    n_exp = gate_weight.shape[0]
    rows_c = tokens // n_chunks
    r = min(_ROWS_PER_BLOCK, rows_c)
    blk0 = chunk * (rows_c // r)
    return pl.pallas_call(
        _matmul_block,
        grid=(rows_c // r,),
        in_specs=[
            pl.BlockSpec((r, dim), lambda b: (b + blk0, 0)),
            pl.BlockSpec((n_exp, dim), lambda b: (0, 0)),
        ],
        out_specs=(
            pl.BlockSpec((r, n_exp), lambda b: (b, 0)),
            pl.BlockSpec((r // 2, 2 * n_exp), lambda b: (b, 0)),
        ),
        out_shape=(
            jax.ShapeDtypeStruct((rows_c, n_exp), jnp.float32),
            jax.ShapeDtypeStruct((rows_c // 2, 2 * n_exp), jnp.float32),
        ),
        compiler_params=pltpu.CompilerParams(
            dimension_semantics=("arbitrary",),
        ),
    )(hidden_states, gate_weight)


def _make_topk_sc(rows_c, r_block):
    half = r_block // 2
    rows_flat = rows_c // 2
    tpw = rows_flat // _N_TILES
    mesh = plsc.VectorSubcoreMesh(core_axis_name="c", subcore_axis_name="s")
    cp = pltpu.CompilerParams()
    if "needs_layout_passes" in pltpu.CompilerParams.__dataclass_fields__:
        cp = dataclasses.replace(cp, needs_layout_passes=False)

    @functools.partial(
        pl.kernel,
        out_type=(
            jax.ShapeDtypeStruct((rows_c * _TOP_K,), jnp.float32),
            jax.ShapeDtypeStruct((rows_c * _TOP_K,), jnp.int32),
        ),
        mesh=mesh,
        scratch_types=[
            pltpu.VMEM((tpw * 128,), jnp.float32),
            pltpu.VMEM((2 * tpw * _TOP_K + 16,), jnp.float32),
            pltpu.VMEM((2 * tpw * _TOP_K + 16,), jnp.int32),
        ],
        compiler_params=cp,
    )
    def topk_kernel(flat_hbm, w_hbm, i_hbm, lv, wv, iv):
        wid = lax.axis_index("s") * _SC_CORES + lax.axis_index("c")
        fbase = wid * tpw
        pltpu.sync_copy(flat_hbm.at[pl.ds(fbase * 128, tpw * 128)], lv)

        blk = fbase // half
        tok_top = blk * r_block + (fbase - blk * half)
        tok_bot = tok_top + half

        lanes = lax.iota(jnp.int32, 16)
        low = lanes < 8

        def merge(ak, av, bk, bv):
            mk = jnp.where(low, ak, lax.rev(bk, (0,)))
            mv = jnp.where(low, av, lax.rev(bv, (0,)))
            return plsc.sort_key_val(mk, mv, descending=True)

        @pl.loop(0, tpw)
        def _row(r):
            rbase = r * 128
            for h in range(2):
                ks, vs = [], []
                for j in range(4):
                    c = lv[pl.ds(rbase + h * 64 + 16 * j, 16)]
                    sk, sv = plsc.sort_key_val(c, lanes + (16 * j),
                                               descending=True)
                    ks.append(sk)
                    vs.append(sv)
                abk, abv = merge(ks[0], vs[0], ks[1], vs[1])
                cdk, cdv = merge(ks[2], vs[2], ks[3], vs[3])
                k8, i8 = merge(abk, abv, cdk, cdv)

                m = jnp.max(k8)
                e = jnp.exp(k8 - m)
                den = jnp.sum(jnp.where(low, e, 0.0))
                w = e / den
                out = (h * tpw + r) * _TOP_K
                plsc.store_compressed(wv.at[pl.ds(out, 16)], w, mask=low)
                plsc.store_compressed(iv.at[pl.ds(out, 16)], i8, mask=low)

        n = tpw * _TOP_K
        pltpu.sync_copy(wv.at[pl.ds(0, n)],
                        w_hbm.at[pl.ds(tok_top * _TOP_K, n)])
        pltpu.sync_copy(iv.at[pl.ds(0, n)],
                        i_hbm.at[pl.ds(tok_top * _TOP_K, n)])
        pltpu.sync_copy(wv.at[pl.ds(n, n)],
                        w_hbm.at[pl.ds(tok_bot * _TOP_K, n)])
        pltpu.sync_copy(iv.at[pl.ds(n, n)],
                        i_hbm.at[pl.ds(tok_bot * _TOP_K, n)])

    return topk_kernel


@jax.jit
def kernel(hidden_states, gate_weight):
    tokens, dim = hidden_states.shape
    rows_c = tokens // _N_CHUNKS
    r = min(_ROWS_PER_BLOCK, rows_c)
    topk = _make_topk_sc(rows_c, r)
    lg_parts, w_parts, i_parts = [], [], []
    for c in range(_N_CHUNKS):
        lg, flat = _matmul_chunk(hidden_states, gate_weight, c, _N_CHUNKS)
        w_f, i_f = topk(flat.reshape(-1))
        lg_parts.append(lg)
        w_parts.append(w_f)
        i_parts.append(i_f)
    logits = jnp.concatenate(lg_parts, axis=0)
    w = jnp.concatenate(w_parts, axis=0).reshape(tokens, _TOP_K)
    i = jnp.concatenate(i_parts, axis=0).reshape(tokens, _TOP_K)
    return (w, i, logits)


# speedup vs baseline: 2.0570x; 1.0300x over previous
import dataclasses
import functools

import jax
import jax.numpy as jnp
from jax import lax
from jax.experimental import pallas as pl
from jax.experimental.pallas import tpu as pltpu
from jax.experimental.pallas import tpu_sc as plsc

_TOP_K = 8
_N_EXP = 64
_ROWS_PER_BLOCK = 1024
_N_TILES = 32
_SC_CORES = 2
_N_CHUNKS = 4


def _matmul_block(hs_ref, gw_ref, logits_ref, flat_ref):
    hs = hs_ref[...]
    gw = gw_ref[...]
    r, dim = hs.shape
    dn = (((1,), (1,)), ((), ()))
    top = jax.lax.dot_general(
        hs[0:r // 2, :], gw, dn, preferred_element_type=jnp.float32)
    bot = jax.lax.dot_general(
        hs[r // 2:r, :], gw, dn, preferred_element_type=jnp.float32)
    logits_ref[...] = jnp.concatenate([top, bot], axis=0)
    flat_ref[...] = jnp.concatenate([top, bot], axis=1)


def _matmul_chunk(hidden_states, gate_weight, chunk, n_chunks):
    tokens, dim = hidden_states.shape
    n_exp = gate_weight.shape[0]
    rows_c = tokens // n_chunks
    r = min(_ROWS_PER_BLOCK, rows_c)
    blk0 = chunk * (rows_c // r)
    return pl.pallas_call(
        _matmul_block,
        grid=(rows_c // r,),
        in_specs=[
            pl.BlockSpec((r, dim), lambda b: (b + blk0, 0)),
            pl.BlockSpec((n_exp, dim), lambda b: (0, 0)),
        ],
        out_specs=(
            pl.BlockSpec((r, n_exp), lambda b: (b, 0)),
            pl.BlockSpec((r // 2, 2 * n_exp), lambda b: (b, 0)),
        ),
        out_shape=(
            jax.ShapeDtypeStruct((rows_c, n_exp), jnp.float32),
            jax.ShapeDtypeStruct((rows_c // 2, 2 * n_exp), jnp.float32),
        ),
        compiler_params=pltpu.CompilerParams(
            dimension_semantics=("arbitrary",),
        ),
    )(hidden_states, gate_weight)


def _make_topk_sc(rows_c, r_block):
    half = r_block // 2
    rows_flat = rows_c // 2
    tpw = rows_flat // _N_TILES
    mesh = plsc.VectorSubcoreMesh(core_axis_name="c", subcore_axis_name="s")
    cp = pltpu.CompilerParams()
    if "needs_layout_passes" in pltpu.CompilerParams.__dataclass_fields__:
        cp = dataclasses.replace(cp, needs_layout_passes=False)

    @functools.partial(
        pl.kernel,
        out_type=(
            jax.ShapeDtypeStruct((rows_c * _TOP_K,), jnp.float32),
            jax.ShapeDtypeStruct((rows_c * _TOP_K,), jnp.int32),
        ),
        mesh=mesh,
        scratch_types=[
            pltpu.VMEM((tpw * 128,), jnp.float32),
            pltpu.VMEM((2 * tpw * _TOP_K + 16,), jnp.float32),
            pltpu.VMEM((2 * tpw * _TOP_K + 16,), jnp.int32),
        ],
        compiler_params=cp,
    )
    def topk_kernel(flat_hbm, w_hbm, i_hbm, lv, wv, iv):
        wid = lax.axis_index("s") * _SC_CORES + lax.axis_index("c")
        fbase = wid * tpw
        pltpu.sync_copy(flat_hbm.at[pl.ds(fbase * 128, tpw * 128)], lv)

        blk = fbase // half
        tok_top = blk * r_block + (fbase - blk * half)
        tok_bot = tok_top + half

        lanes = lax.iota(jnp.int32, 16)
        low = lanes < 8

        def merge(ak, av, bk, bv):
            mk = jnp.where(low, ak, lax.rev(bk, (0,)))
            mv = jnp.where(low, av, lax.rev(bv, (0,)))
            return plsc.sort_key_val(mk, mv, descending=True)

        @plsc.parallel_loop(0, tpw, 1, unroll=2)
        def _row(r):
            rbase = r * 128
            for h in range(2):
                ks, vs = [], []
                for j in range(4):
                    c = lv[pl.ds(rbase + h * 64 + 16 * j, 16)]
                    sk, sv = plsc.sort_key_val(c, lanes + (16 * j),
                                               descending=True)
                    ks.append(sk)
                    vs.append(sv)
                abk, abv = merge(ks[0], vs[0], ks[1], vs[1])
                cdk, cdv = merge(ks[2], vs[2], ks[3], vs[3])
                k8, i8 = merge(abk, abv, cdk, cdv)

                m = jnp.max(k8)
                e = jnp.exp(k8 - m)
                den = jnp.sum(jnp.where(low, e, 0.0))
                w = e / den
                out = (h * tpw + r) * _TOP_K
                plsc.store_compressed(wv.at[pl.ds(out, 16)], w, mask=low)
                plsc.store_compressed(iv.at[pl.ds(out, 16)], i8, mask=low)

        n = tpw * _TOP_K
        pltpu.sync_copy(wv.at[pl.ds(0, n)],
                        w_hbm.at[pl.ds(tok_top * _TOP_K, n)])
        pltpu.sync_copy(iv.at[pl.ds(0, n)],
                        i_hbm.at[pl.ds(tok_top * _TOP_K, n)])
        pltpu.sync_copy(wv.at[pl.ds(n, n)],
                        w_hbm.at[pl.ds(tok_bot * _TOP_K, n)])
        pltpu.sync_copy(iv.at[pl.ds(n, n)],
                        i_hbm.at[pl.ds(tok_bot * _TOP_K, n)])

    return topk_kernel


@jax.jit
def kernel(hidden_states, gate_weight):
    tokens, dim = hidden_states.shape
    rows_c = tokens // _N_CHUNKS
    r = min(_ROWS_PER_BLOCK, rows_c)
    topk = _make_topk_sc(rows_c, r)
    lg_parts, w_parts, i_parts = [], [], []
    for c in range(_N_CHUNKS):
        lg, flat = _matmul_chunk(hidden_states, gate_weight, c, _N_CHUNKS)
        w_f, i_f = topk(flat.reshape(-1))
        lg_parts.append(lg)
        w_parts.append(w_f)
        i_parts.append(i_f)
    logits = jnp.concatenate(lg_parts, axis=0)
    w2 = jnp.concatenate(
        [p.reshape(rows_c * _TOP_K // 128, 128) for p in w_parts], axis=0)
    i2 = jnp.concatenate(
        [p.reshape(rows_c * _TOP_K // 128, 128) for p in i_parts], axis=0)
    w2, i2 = jax.lax.optimization_barrier((w2, i2))
    w = w2.reshape(tokens, _TOP_K)
    i = i2.reshape(tokens, _TOP_K)
    return (w, i, logits)


# speedup vs baseline: 2.0699x; 1.0063x over previous
import dataclasses
import functools

import jax
import jax.numpy as jnp
from jax import lax
from jax.experimental import pallas as pl
from jax.experimental.pallas import tpu as pltpu
from jax.experimental.pallas import tpu_sc as plsc

_TOP_K = 8
_N_EXP = 64
_ROWS_PER_BLOCK = 1024
_N_TILES = 32
_SC_CORES = 2
_N_CHUNKS = 4


def _matmul_block_first(hs_ref, gw_ref, logits_ref, flat_ref):
    _matmul_body(hs_ref, gw_ref, logits_ref, flat_ref)


def _matmul_block(hs_ref, gw_ref, prev_ref, logits_ref, flat_ref):
    del prev_ref
    _matmul_body(hs_ref, gw_ref, logits_ref, flat_ref)


def _matmul_body(hs_ref, gw_ref, logits_ref, flat_ref):
    hs = hs_ref[...]
    gw = gw_ref[...]
    r, dim = hs.shape
    dn = (((1,), (1,)), ((), ()))
    top = jax.lax.dot_general(
        hs[0:r // 2, :], gw, dn, preferred_element_type=jnp.float32)
    bot = jax.lax.dot_general(
        hs[r // 2:r, :], gw, dn, preferred_element_type=jnp.float32)
    logits_ref[...] = jnp.concatenate([top, bot], axis=0)
    flat_ref[...] = jnp.concatenate([top, bot], axis=1)


def _matmul_chunk(hidden_states, gate_weight, prev_logits, chunk, n_chunks):
    tokens, dim = hidden_states.shape
    n_exp = gate_weight.shape[0]
    rows_c = tokens // n_chunks
    r = min(_ROWS_PER_BLOCK, rows_c)
    blk0 = chunk * (rows_c // r)
    in_specs = [
        pl.BlockSpec((r, dim), lambda b: (b + blk0, 0)),
        pl.BlockSpec((n_exp, dim), lambda b: (0, 0)),
    ]
    operands = [hidden_states, gate_weight]
    if prev_logits is None:
        body = _matmul_block_first
        aliases = {}
    else:
        body = _matmul_block
        in_specs.append(pl.BlockSpec(memory_space=pltpu.MemorySpace.HBM))
        operands.append(prev_logits)
        aliases = {2: 0}
    return pl.pallas_call(
        body,
        grid=(rows_c // r,),
        in_specs=in_specs,
        out_specs=(
            pl.BlockSpec((r, n_exp), lambda b: (b + blk0, 0)),
            pl.BlockSpec((r // 2, 2 * n_exp), lambda b: (b, 0)),
        ),
        out_shape=(
            jax.ShapeDtypeStruct((tokens, n_exp), jnp.float32),
            jax.ShapeDtypeStruct((rows_c // 2, 2 * n_exp), jnp.float32),
        ),
        input_output_aliases=aliases,
        compiler_params=pltpu.CompilerParams(
            dimension_semantics=("arbitrary",),
        ),
    )(*operands)


def _make_topk_sc(rows_c, r_block):
    half = r_block // 2
    rows_flat = rows_c // 2
    tpw = rows_flat // _N_TILES
    mesh = plsc.VectorSubcoreMesh(core_axis_name="c", subcore_axis_name="s")
    cp = pltpu.CompilerParams()
    if "needs_layout_passes" in pltpu.CompilerParams.__dataclass_fields__:
        cp = dataclasses.replace(cp, needs_layout_passes=False)

    @functools.partial(
        pl.kernel,
        out_type=(
            jax.ShapeDtypeStruct((rows_c * _TOP_K,), jnp.float32),
            jax.ShapeDtypeStruct((rows_c * _TOP_K,), jnp.int32),
        ),
        mesh=mesh,
        scratch_types=[
            pltpu.VMEM((tpw * 128,), jnp.float32),
            pltpu.VMEM((2 * tpw * _TOP_K + 16,), jnp.float32),
            pltpu.VMEM((2 * tpw * _TOP_K + 16,), jnp.int32),
        ],
        compiler_params=cp,
    )
    def topk_kernel(flat_hbm, w_hbm, i_hbm, lv, wv, iv):
        wid = lax.axis_index("s") * _SC_CORES + lax.axis_index("c")
        fbase = wid * tpw
        pltpu.sync_copy(flat_hbm.at[pl.ds(fbase * 128, tpw * 128)], lv)

        blk = fbase // half
        tok_top = blk * r_block + (fbase - blk * half)
        tok_bot = tok_top + half

        lanes = lax.iota(jnp.int32, 16)
        low = lanes < 8

        def merge(ak, av, bk, bv):
            mk = jnp.where(low, ak, lax.rev(bk, (0,)))
            mv = jnp.where(low, av, lax.rev(bv, (0,)))
            return plsc.sort_key_val(mk, mv, descending=True)

        @plsc.parallel_loop(0, tpw, 1, unroll=2)
        def _row(r):
            rbase = r * 128
            for h in range(2):
                ks, vs = [], []
                for j in range(4):
                    c = lv[pl.ds(rbase + h * 64 + 16 * j, 16)]
                    sk, sv = plsc.sort_key_val(c, lanes + (16 * j),
                                               descending=True)
                    ks.append(sk)
                    vs.append(sv)
                abk, abv = merge(ks[0], vs[0], ks[1], vs[1])
                cdk, cdv = merge(ks[2], vs[2], ks[3], vs[3])
                k8, i8 = merge(abk, abv, cdk, cdv)

                m = jnp.max(k8)
                e = jnp.exp(k8 - m)
                den = jnp.sum(jnp.where(low, e, 0.0))
                w = e / den
                out = (h * tpw + r) * _TOP_K
                plsc.store_compressed(wv.at[pl.ds(out, 16)], w, mask=low)
                plsc.store_compressed(iv.at[pl.ds(out, 16)], i8, mask=low)

        n = tpw * _TOP_K
        pltpu.sync_copy(wv.at[pl.ds(0, n)],
                        w_hbm.at[pl.ds(tok_top * _TOP_K, n)])
        pltpu.sync_copy(iv.at[pl.ds(0, n)],
                        i_hbm.at[pl.ds(tok_top * _TOP_K, n)])
        pltpu.sync_copy(wv.at[pl.ds(n, n)],
                        w_hbm.at[pl.ds(tok_bot * _TOP_K, n)])
        pltpu.sync_copy(iv.at[pl.ds(n, n)],
                        i_hbm.at[pl.ds(tok_bot * _TOP_K, n)])

    return topk_kernel


@jax.jit
def kernel(hidden_states, gate_weight):
    tokens, dim = hidden_states.shape
    rows_c = tokens // _N_CHUNKS
    r = min(_ROWS_PER_BLOCK, rows_c)
    topk = _make_topk_sc(rows_c, r)
    w_parts, i_parts = [], []
    logits = None
    for c in range(_N_CHUNKS):
        logits, flat = _matmul_chunk(hidden_states, gate_weight, logits,
                                     c, _N_CHUNKS)
        w_f, i_f = topk(flat.reshape(-1))
        w_parts.append(w_f)
        i_parts.append(i_f)
    w2 = jnp.concatenate(
        [p.reshape(rows_c * _TOP_K // 128, 128) for p in w_parts], axis=0)
    i2 = jnp.concatenate(
        [p.reshape(rows_c * _TOP_K // 128, 128) for p in i_parts], axis=0)
    w2, i2 = jax.lax.optimization_barrier((w2, i2))
    w = w2.reshape(tokens, _TOP_K)
    i = i2.reshape(tokens, _TOP_K)
    return (w, i, logits)


# speedup vs baseline: 2.2820x; 1.1025x over previous
import dataclasses
import functools

import jax
import jax.numpy as jnp
from jax import lax
from jax.experimental import pallas as pl
from jax.experimental.pallas import tpu as pltpu
from jax.experimental.pallas import tpu_sc as plsc

_TOP_K = 8
_N_EXP = 64
_ROWS_PER_BLOCK = 1024
_N_TILES = 32
_SC_CORES = 2
_N_CHUNKS = 2


def _matmul_block_first(hs_ref, gw_ref, logits_ref, flat_ref):
    _matmul_body(hs_ref, gw_ref, logits_ref, flat_ref)


def _matmul_block(hs_ref, gw_ref, prev_ref, logits_ref, flat_ref):
    del prev_ref
    _matmul_body(hs_ref, gw_ref, logits_ref, flat_ref)


def _matmul_body(hs_ref, gw_ref, logits_ref, flat_ref):
    hs = hs_ref[...]
    gw = gw_ref[...]
    r, dim = hs.shape
    dn = (((1,), (1,)), ((), ()))
    top = jax.lax.dot_general(
        hs[0:r // 2, :], gw, dn, preferred_element_type=jnp.float32)
    bot = jax.lax.dot_general(
        hs[r // 2:r, :], gw, dn, preferred_element_type=jnp.float32)
    logits_ref[...] = jnp.concatenate([top, bot], axis=0)
    flat_ref[...] = jnp.concatenate([top, bot], axis=1)


def _matmul_chunk(hidden_states, gate_weight, prev_logits, chunk, n_chunks):
    tokens, dim = hidden_states.shape
    n_exp = gate_weight.shape[0]
    rows_c = tokens // n_chunks
    r = min(_ROWS_PER_BLOCK, rows_c)
    blk0 = chunk * (rows_c // r)
    in_specs = [
        pl.BlockSpec((r, dim), lambda b: (b + blk0, 0)),
        pl.BlockSpec((n_exp, dim), lambda b: (0, 0)),
    ]
    operands = [hidden_states, gate_weight]
    if prev_logits is None:
        body = _matmul_block_first
        aliases = {}
    else:
        body = _matmul_block
        in_specs.append(pl.BlockSpec(memory_space=pltpu.MemorySpace.HBM))
        operands.append(prev_logits)
        aliases = {2: 0}
    return pl.pallas_call(
        body,
        grid=(rows_c // r,),
        in_specs=in_specs,
        out_specs=(
            pl.BlockSpec((r, n_exp), lambda b: (b + blk0, 0)),
            pl.BlockSpec((r // 2, 2 * n_exp), lambda b: (b, 0)),
        ),
        out_shape=(
            jax.ShapeDtypeStruct((tokens, n_exp), jnp.float32),
            jax.ShapeDtypeStruct((rows_c // 2, 2 * n_exp), jnp.float32),
        ),
        input_output_aliases=aliases,
        compiler_params=pltpu.CompilerParams(
            dimension_semantics=("arbitrary",),
        ),
    )(*operands)


def _make_topk_sc(rows_c, r_block):
    half = r_block // 2
    rows_flat = rows_c // 2
    tpw = rows_flat // _N_TILES
    mesh = plsc.VectorSubcoreMesh(core_axis_name="c", subcore_axis_name="s")
    cp = pltpu.CompilerParams()
    if "needs_layout_passes" in pltpu.CompilerParams.__dataclass_fields__:
        cp = dataclasses.replace(cp, needs_layout_passes=False)

    @functools.partial(
        pl.kernel,
        out_type=(
            jax.ShapeDtypeStruct((rows_c * _TOP_K,), jnp.float32),
            jax.ShapeDtypeStruct((rows_c * _TOP_K,), jnp.int32),
        ),
        mesh=mesh,
        scratch_types=[
            pltpu.VMEM((tpw * 128,), jnp.float32),
            pltpu.VMEM((2 * tpw * _TOP_K + 16,), jnp.float32),
            pltpu.VMEM((2 * tpw * _TOP_K + 16,), jnp.int32),
        ],
        compiler_params=cp,
    )
    def topk_kernel(flat_hbm, w_hbm, i_hbm, lv, wv, iv):
        wid = lax.axis_index("s") * _SC_CORES + lax.axis_index("c")
        fbase = wid * tpw
        pltpu.sync_copy(flat_hbm.at[pl.ds(fbase * 128, tpw * 128)], lv)

        blk = fbase // half
        tok_top = blk * r_block + (fbase - blk * half)
        tok_bot = tok_top + half

        lanes = lax.iota(jnp.int32, 16)
        low = lanes < 8

        def merge(ak, av, bk, bv):
            mk = jnp.where(low, ak, lax.rev(bk, (0,)))
            mv = jnp.where(low, av, lax.rev(bv, (0,)))
            return plsc.sort_key_val(mk, mv, descending=True)

        @plsc.parallel_loop(0, tpw, 1, unroll=2)
        def _row(r):
            rbase = r * 128
            for h in range(2):
                ks, vs = [], []
                for j in range(4):
                    c = lv[pl.ds(rbase + h * 64 + 16 * j, 16)]
                    sk, sv = plsc.sort_key_val(c, lanes + (16 * j),
                                               descending=True)
                    ks.append(sk)
                    vs.append(sv)
                abk, abv = merge(ks[0], vs[0], ks[1], vs[1])
                cdk, cdv = merge(ks[2], vs[2], ks[3], vs[3])
                k8, i8 = merge(abk, abv, cdk, cdv)

                m = jnp.max(k8)
                e = jnp.exp(k8 - m)
                den = jnp.sum(jnp.where(low, e, 0.0))
                w = e / den
                out = (h * tpw + r) * _TOP_K
                plsc.store_compressed(wv.at[pl.ds(out, 16)], w, mask=low)
                plsc.store_compressed(iv.at[pl.ds(out, 16)], i8, mask=low)

        n = tpw * _TOP_K
        pltpu.sync_copy(wv.at[pl.ds(0, n)],
                        w_hbm.at[pl.ds(tok_top * _TOP_K, n)])
        pltpu.sync_copy(iv.at[pl.ds(0, n)],
                        i_hbm.at[pl.ds(tok_top * _TOP_K, n)])
        pltpu.sync_copy(wv.at[pl.ds(n, n)],
                        w_hbm.at[pl.ds(tok_bot * _TOP_K, n)])
        pltpu.sync_copy(iv.at[pl.ds(n, n)],
                        i_hbm.at[pl.ds(tok_bot * _TOP_K, n)])

    return topk_kernel


@jax.jit
def kernel(hidden_states, gate_weight):
    tokens, dim = hidden_states.shape
    rows_c = tokens // _N_CHUNKS
    r = min(_ROWS_PER_BLOCK, rows_c)
    topk = _make_topk_sc(rows_c, r)
    w_parts, i_parts = [], []
    logits = None
    for c in range(_N_CHUNKS):
        logits, flat = _matmul_chunk(hidden_states, gate_weight, logits,
                                     c, _N_CHUNKS)
        w_f, i_f = topk(flat.reshape(-1))
        w_parts.append(w_f)
        i_parts.append(i_f)
    w2 = jnp.concatenate(
        [p.reshape(rows_c * _TOP_K // 128, 128) for p in w_parts], axis=0)
    i2 = jnp.concatenate(
        [p.reshape(rows_c * _TOP_K // 128, 128) for p in i_parts], axis=0)
    w2, i2 = jax.lax.optimization_barrier((w2, i2))
    w = w2.reshape(tokens, _TOP_K)
    i = i2.reshape(tokens, _TOP_K)
    return (w, i, logits)
